# Initial kernel scaffold; baseline (speedup 1.0000x reference)
import jax
import jax.numpy as jnp
from jax.experimental import pallas as pl


def kernel(x, table, W_h, b_h, W_o, b_o):
    raise NotImplementedError("write your pallas kernel here")


# speedup vs baseline: 2.2701x; 2.2701x over previous
import functools

import jax
import jax.numpy as jnp
from jax import lax
from jax.experimental import pallas as pl
from jax.experimental.pallas import tpu as pltpu
from jax.experimental.pallas import tpu_sc as plsc

VOCAB = 5231
EMB = 300
D_PAD = 304
OUT_DIM = 37
BATCH = 16384
SEQ = 2
NIDX = BATCH * SEQ

_NC = 2
_NS = 16
_NW = _NC * _NS
_B_PER_W = NIDX // _NW
_CHUNK = 256


def _sc_gather_body(table_hbm, idx_hbm, out_hbm, idx_v, rows_v, sem):
    wid = lax.axis_index("s") * _NC + lax.axis_index("c")
    base = wid * _B_PER_W
    pltpu.sync_copy(idx_hbm.at[pl.ds(base, _B_PER_W)], idx_v)
    for c in range(_B_PER_W // _CHUNK):
        pltpu.async_copy(
            table_hbm.at[idx_v.at[pl.ds(c * _CHUNK, _CHUNK)]], rows_v, sem
        ).wait()
        pltpu.sync_copy(rows_v, out_hbm.at[pl.ds(base + c * _CHUNK, _CHUNK)])


_sc_gather = functools.partial(
    pl.kernel,
    out_type=jax.ShapeDtypeStruct((NIDX, D_PAD), jnp.float32),
    mesh=plsc.VectorSubcoreMesh(core_axis_name="c", subcore_axis_name="s"),
    scratch_types=[
        pltpu.VMEM((_B_PER_W,), jnp.int32),
        pltpu.VMEM((_CHUNK, D_PAD), jnp.float32),
        pltpu.SemaphoreType.DMA,
    ],
    compiler_params=pltpu.CompilerParams(use_tc_tiling_on_sc=False),
)(_sc_gather_body)


def _mlp_body(flat_ref, wh_ref, bh_ref, wo_ref, bo_ref, out_ref):
    flat = flat_ref[...]
    h = jnp.dot(flat, wh_ref[...], preferred_element_type=jnp.float32)
    h = jax.nn.sigmoid(h + bh_ref[...])
    logits = jnp.dot(h, wo_ref[...], preferred_element_type=jnp.float32)
    logits = logits + bo_ref[...]
    m = jnp.max(logits, axis=-1, keepdims=True)
    s = logits - m
    lse = jnp.log(jnp.sum(jnp.exp(s), axis=-1, keepdims=True))
    out_ref[...] = s - lse


_TB = 1024


def _mlp(flat, wh, bh, wo, bo):
    grid = BATCH // _TB
    return pl.pallas_call(
        _mlp_body,
        grid=(grid,),
        in_specs=[
            pl.BlockSpec((_TB, 2 * D_PAD), lambda i: (i, 0)),
            pl.BlockSpec((2 * D_PAD, EMB), lambda i: (0, 0)),
            pl.BlockSpec((1, EMB), lambda i: (0, 0)),
            pl.BlockSpec((EMB, OUT_DIM), lambda i: (0, 0)),
            pl.BlockSpec((1, OUT_DIM), lambda i: (0, 0)),
        ],
        out_specs=pl.BlockSpec((_TB, OUT_DIM), lambda i: (i, 0)),
        out_shape=jax.ShapeDtypeStruct((BATCH, OUT_DIM), jnp.float32),
    )(flat, wh, bh, wo, bo)


def kernel(x, table, W_h, b_h, W_o, b_o):
    idx = x.reshape(NIDX).astype(jnp.int32)
    table_pad = jnp.pad(table, ((0, 0), (0, D_PAD - EMB)))
    e_pad = _sc_gather(table_pad, idx)
    flat = e_pad.reshape(BATCH, 2 * D_PAD)
    wh_pad = jnp.zeros((2 * D_PAD, EMB), jnp.float32)
    wh_pad = wh_pad.at[0:EMB].set(W_h[0:EMB])
    wh_pad = wh_pad.at[D_PAD : D_PAD + EMB].set(W_h[EMB:])
    return _mlp(flat, wh_pad, b_h.reshape(1, EMB), W_o, b_o.reshape(1, OUT_DIM))


# speedup vs baseline: 3.2024x; 1.4107x over previous
import functools

import jax
import jax.numpy as jnp
from jax import lax
from jax.experimental import pallas as pl
from jax.experimental.pallas import tpu as pltpu
from jax.experimental.pallas import tpu_sc as plsc

VOCAB = 5231
EMB = 300
D_PAD = 304
OUT_DIM = 37
BATCH = 16384
SEQ = 2

_NC = 2
_NS = 16
_NW = _NC * _NS
_B_PER_W = BATCH // _NW
_CHUNK = 256

_VT = 512


def _precompute_body(table_ref, wh_ref, bh_ref, t0_ref, t1_ref):
    t = table_ref[...]
    z = jnp.zeros((t.shape[0], D_PAD - EMB), jnp.float32)
    a = jnp.dot(t, wh_ref[0:EMB, :], preferred_element_type=jnp.float32)
    t0_ref[...] = jnp.concatenate([a + bh_ref[...], z], axis=1)
    b = jnp.dot(t, wh_ref[EMB:, :], preferred_element_type=jnp.float32)
    t1_ref[...] = jnp.concatenate([b, z], axis=1)


def _precompute(table, wh, bh):
    grid = (VOCAB + _VT - 1) // _VT
    return pl.pallas_call(
        _precompute_body,
        grid=(grid,),
        in_specs=[
            pl.BlockSpec((_VT, EMB), lambda i: (i, 0)),
            pl.BlockSpec((2 * EMB, EMB), lambda i: (0, 0)),
            pl.BlockSpec((1, EMB), lambda i: (0, 0)),
        ],
        out_specs=[
            pl.BlockSpec((_VT, D_PAD), lambda i: (i, 0)),
            pl.BlockSpec((_VT, D_PAD), lambda i: (i, 0)),
        ],
        out_shape=[
            jax.ShapeDtypeStruct((VOCAB, D_PAD), jnp.float32),
            jax.ShapeDtypeStruct((VOCAB, D_PAD), jnp.float32),
        ],
    )(table, wh, bh)


def _sc_gather_body(t0_hbm, t1_hbm, idx0_hbm, idx1_hbm, out_hbm, i0_v, i1_v, rows_v, sem):
    wid = lax.axis_index("s") * _NC + lax.axis_index("c")
    base = wid * _B_PER_W
    pltpu.sync_copy(idx0_hbm.at[pl.ds(base, _B_PER_W)], i0_v)
    pltpu.sync_copy(idx1_hbm.at[pl.ds(base, _B_PER_W)], i1_v)
    for c in range(_B_PER_W // _CHUNK):
        pltpu.async_copy(
            t0_hbm.at[i0_v.at[pl.ds(c * _CHUNK, _CHUNK)]], rows_v, sem
        ).wait()
        pltpu.async_copy(
            t1_hbm.at[i1_v.at[pl.ds(c * _CHUNK, _CHUNK)]], rows_v, sem, add=True
        ).wait()
        pltpu.sync_copy(rows_v, out_hbm.at[pl.ds(base + c * _CHUNK, _CHUNK)])


_sc_gather = functools.partial(
    pl.kernel,
    out_type=jax.ShapeDtypeStruct((BATCH, D_PAD), jnp.float32),
    mesh=plsc.VectorSubcoreMesh(core_axis_name="c", subcore_axis_name="s"),
    scratch_types=[
        pltpu.VMEM((_B_PER_W,), jnp.int32),
        pltpu.VMEM((_B_PER_W,), jnp.int32),
        pltpu.VMEM((_CHUNK, D_PAD), jnp.float32),
        pltpu.SemaphoreType.DMA,
    ],
    compiler_params=pltpu.CompilerParams(use_tc_tiling_on_sc=False),
)(_sc_gather_body)


def _mlp_body(hp_ref, wo_ref, bo_ref, out_ref):
    h = jax.nn.sigmoid(hp_ref[...][:, 0:EMB])
    logits = jnp.dot(h, wo_ref[...], preferred_element_type=jnp.float32)
    logits = logits + bo_ref[...]
    m = jnp.max(logits, axis=-1, keepdims=True)
    s = logits - m
    lse = jnp.log(jnp.sum(jnp.exp(s), axis=-1, keepdims=True))
    out_ref[...] = s - lse


_TB = 1024


def _mlp(hp, wo, bo):
    grid = BATCH // _TB
    return pl.pallas_call(
        _mlp_body,
        grid=(grid,),
        in_specs=[
            pl.BlockSpec((_TB, D_PAD), lambda i: (i, 0)),
            pl.BlockSpec((EMB, OUT_DIM), lambda i: (0, 0)),
            pl.BlockSpec((1, OUT_DIM), lambda i: (0, 0)),
        ],
        out_specs=pl.BlockSpec((_TB, OUT_DIM), lambda i: (i, 0)),
        out_shape=jax.ShapeDtypeStruct((BATCH, OUT_DIM), jnp.float32),
    )(hp, wo, bo)


def kernel(x, table, W_h, b_h, W_o, b_o):
    xi = x.astype(jnp.int32)
    idx0 = xi[:, 0]
    idx1 = xi[:, 1]
    t0, t1 = _precompute(table, W_h, b_h.reshape(1, EMB))
    hp = _sc_gather(t0, t1, idx0, idx1)
    return _mlp(hp, W_o, b_o.reshape(1, OUT_DIM))


# speedup vs baseline: 4.3580x; 1.3608x over previous
import functools

import jax
import jax.numpy as jnp
from jax import lax
from jax.experimental import pallas as pl
from jax.experimental.pallas import tpu as pltpu
from jax.experimental.pallas import tpu_sc as plsc

VOCAB = 5231
VPAD = 5232
EMB = 300
OUT_DIM = 37
BATCH = 16384
SEQ = 2
NSLAB = 3

_NC = 2
_NS = 16
_NW = _NC * _NS
_B_PER_W = BATCH // _NW

_VT = 1744


def _precompute_body(table_ref, wh_ref, bh_ref, t0_ref, t1_ref):
    k = pl.program_id(1)
    t = table_ref[...]

    def slab(w_full, bias, lo, width):
        c = jnp.dot(
            t, w_full[:, lo : lo + width], preferred_element_type=jnp.float32
        )
        if bias is not None:
            c = c + bias[:, lo : lo + width]
        if width < 128:
            c = jnp.concatenate(
                [c, jnp.zeros((c.shape[0], 128 - width), jnp.float32)], axis=1
            )
        return c

    wa = wh_ref[0:EMB, :]
    wb = wh_ref[EMB:, :]
    for kk in range(NSLAB):
        lo = 128 * kk
        width = min(128, EMB - lo)

        @pl.when(k == kk)
        def _():
            t0_ref[...] = slab(wa, bh_ref, lo, width)
            t1_ref[...] = slab(wb, None, lo, width)


def _precompute(table, wh, bh):
    grid = (VPAD // _VT, NSLAB)
    return pl.pallas_call(
        _precompute_body,
        grid=grid,
        in_specs=[
            pl.BlockSpec((_VT, EMB), lambda i, k: (i, 0)),
            pl.BlockSpec((2 * EMB, EMB), lambda i, k: (0, 0)),
            pl.BlockSpec((1, EMB), lambda i, k: (0, 0)),
        ],
        out_specs=[
            pl.BlockSpec((_VT, 128), lambda i, k: (k * (VPAD // _VT) + i, 0)),
            pl.BlockSpec((_VT, 128), lambda i, k: (k * (VPAD // _VT) + i, 0)),
        ],
        out_shape=[
            jax.ShapeDtypeStruct((NSLAB * VPAD, 128), jnp.float32),
            jax.ShapeDtypeStruct((NSLAB * VPAD, 128), jnp.float32),
        ],
    )(table, wh, bh)


def _sc_gather_body(
    t0_hbm, t1_hbm, idx0_hbm, idx1_hbm, out_hbm, i0_v, i1_v, i0s_v, i1s_v, rows_v, sem
):
    wid = lax.axis_index("s") * _NC + lax.axis_index("c")
    base = wid * _B_PER_W
    pltpu.sync_copy(idx0_hbm.at[pl.ds(base, _B_PER_W)], i0_v)
    pltpu.sync_copy(idx1_hbm.at[pl.ds(base, _B_PER_W)], i1_v)
    for k in range(NSLAB):
        shift = jnp.int32(VPAD * k)
        for j in range(_B_PER_W // 16):
            sl = pl.ds(16 * j, 16)
            i0s_v[sl] = i0_v[sl] + shift
            i1s_v[sl] = i1_v[sl] + shift
        pltpu.async_copy(t0_hbm.at[i0s_v], rows_v, sem).wait()
        pltpu.async_copy(t1_hbm.at[i1s_v], rows_v, sem, add=True).wait()
        pltpu.sync_copy(rows_v, out_hbm.at[pl.ds(k * BATCH + base, _B_PER_W)])


_sc_gather = functools.partial(
    pl.kernel,
    out_type=jax.ShapeDtypeStruct((NSLAB * BATCH, 128), jnp.float32),
    mesh=plsc.VectorSubcoreMesh(core_axis_name="c", subcore_axis_name="s"),
    scratch_types=[
        pltpu.VMEM((_B_PER_W,), jnp.int32),
        pltpu.VMEM((_B_PER_W,), jnp.int32),
        pltpu.VMEM((_B_PER_W,), jnp.int32),
        pltpu.VMEM((_B_PER_W,), jnp.int32),
        pltpu.VMEM((_B_PER_W, 128), jnp.float32),
        pltpu.SemaphoreType.DMA,
    ],
)(_sc_gather_body)


def _mlp_body(hp_ref, wo_ref, bo_ref, out_ref):
    s0 = jax.nn.sigmoid(hp_ref[0])
    s1 = jax.nn.sigmoid(hp_ref[1])
    s2 = jax.nn.sigmoid(hp_ref[2])[:, 0 : EMB - 256]
    wo = wo_ref[...]
    logits = (
        jnp.dot(s0, wo[0:128, :], preferred_element_type=jnp.float32)
        + jnp.dot(s1, wo[128:256, :], preferred_element_type=jnp.float32)
        + jnp.dot(s2, wo[256:EMB, :], preferred_element_type=jnp.float32)
        + bo_ref[...]
    )
    m = jnp.max(logits, axis=-1, keepdims=True)
    s = logits - m
    lse = jnp.log(jnp.sum(jnp.exp(s), axis=-1, keepdims=True))
    out_ref[...] = s - lse


_TB = 1024


def _mlp(hp, wo, bo):
    grid = BATCH // _TB
    return pl.pallas_call(
        _mlp_body,
        grid=(grid,),
        in_specs=[
            pl.BlockSpec((NSLAB, _TB, 128), lambda i: (0, i, 0)),
            pl.BlockSpec((EMB, OUT_DIM), lambda i: (0, 0)),
            pl.BlockSpec((1, OUT_DIM), lambda i: (0, 0)),
        ],
        out_specs=pl.BlockSpec((_TB, OUT_DIM), lambda i: (i, 0)),
        out_shape=jax.ShapeDtypeStruct((BATCH, OUT_DIM), jnp.float32),
    )(hp, wo, bo)


def kernel(x, table, W_h, b_h, W_o, b_o):
    xi = x.astype(jnp.int32)
    idx0 = xi[:, 0]
    idx1 = xi[:, 1]
    t0p, t1p = _precompute(table, W_h, b_h.reshape(1, EMB))
    hp = _sc_gather(t0p, t1p, idx0, idx1)
    hp3 = hp.reshape(NSLAB, BATCH, 128)
    return _mlp(hp3, W_o, b_o.reshape(1, OUT_DIM))


# speedup vs baseline: 4.7889x; 1.0989x over previous
import functools

import jax
import jax.numpy as jnp
from jax import lax
from jax.experimental import pallas as pl
from jax.experimental.pallas import tpu as pltpu
from jax.experimental.pallas import tpu_sc as plsc

VOCAB = 5231
VPAD = 5232
EMB = 300
OUT_DIM = 37
BATCH = 16384
SEQ = 2
NSLAB = 3

_NC = 2
_NS = 16
_NW = _NC * _NS
_B_PER_W = BATCH // _NW

_VT = 1744


def _precompute_body(table_ref, wh_ref, bh_ref, *t_refs):
    t = table_ref[...]

    def slab(w_full, bias, lo, width):
        c = jnp.dot(
            t, w_full[:, lo : lo + width], preferred_element_type=jnp.float32
        )
        if bias is not None:
            c = c + bias[:, lo : lo + width]
        if width < 128:
            c = jnp.concatenate(
                [c, jnp.zeros((c.shape[0], 128 - width), jnp.float32)], axis=1
            )
        return c

    wa = wh_ref[0:EMB, :]
    wb = wh_ref[EMB:, :]
    for kk in range(NSLAB):
        lo = 128 * kk
        width = min(128, EMB - lo)
        t_refs[kk][...] = slab(wa, bh_ref, lo, width)
        t_refs[NSLAB + kk][...] = slab(wb, None, lo, width)


def _precompute(table, wh, bh):
    grid = (VPAD // _VT,)
    return pl.pallas_call(
        _precompute_body,
        grid=grid,
        in_specs=[
            pl.BlockSpec((_VT, EMB), lambda i: (i, 0)),
            pl.BlockSpec((2 * EMB, EMB), lambda i: (0, 0)),
            pl.BlockSpec((1, EMB), lambda i: (0, 0)),
        ],
        out_specs=[pl.BlockSpec((_VT, 128), lambda i: (i, 0))] * (2 * NSLAB),
        out_shape=[jax.ShapeDtypeStruct((VPAD, 128), jnp.float32)] * (2 * NSLAB),
    )(table, wh, bh)


def _sc_gather_body(
    t00, t01, t02, t10, t11, t12, idx0_hbm, idx1_hbm, out_hbm, i0_v, i1_v, rows_v, sem
):
    wid = lax.axis_index("s") * _NC + lax.axis_index("c")
    base = wid * _B_PER_W
    pltpu.sync_copy(idx0_hbm.at[pl.ds(base, _B_PER_W)], i0_v)
    pltpu.sync_copy(idx1_hbm.at[pl.ds(base, _B_PER_W)], i1_v)
    t0s = (t00, t01, t02)
    t1s = (t10, t11, t12)
    for k in range(NSLAB):
        pltpu.async_copy(t0s[k].at[i0_v], rows_v, sem).wait()
        pltpu.async_copy(t1s[k].at[i1_v], rows_v, sem, add=True).wait()
        pltpu.sync_copy(rows_v, out_hbm.at[pl.ds(k * BATCH + base, _B_PER_W)])


_sc_gather = functools.partial(
    pl.kernel,
    out_type=jax.ShapeDtypeStruct((NSLAB * BATCH, 128), jnp.float32),
    mesh=plsc.VectorSubcoreMesh(core_axis_name="c", subcore_axis_name="s"),
    scratch_types=[
        pltpu.VMEM((_B_PER_W,), jnp.int32),
        pltpu.VMEM((_B_PER_W,), jnp.int32),
        pltpu.VMEM((_B_PER_W, 128), jnp.float32),
        pltpu.SemaphoreType.DMA,
    ],
)(_sc_gather_body)


def _mlp_body(hp_ref, wo_ref, bo_ref, out_ref):
    s0 = jax.nn.sigmoid(hp_ref[0])
    s1 = jax.nn.sigmoid(hp_ref[1])
    s2 = jax.nn.sigmoid(hp_ref[2])[:, 0 : EMB - 256]
    wo = wo_ref[...]
    logits = (
        jnp.dot(s0, wo[0:128, :], preferred_element_type=jnp.float32)
        + jnp.dot(s1, wo[128:256, :], preferred_element_type=jnp.float32)
        + jnp.dot(s2, wo[256:EMB, :], preferred_element_type=jnp.float32)
        + bo_ref[...]
    )
    m = jnp.max(logits, axis=-1, keepdims=True)
    s = logits - m
    lse = jnp.log(jnp.sum(jnp.exp(s), axis=-1, keepdims=True))
    out_ref[...] = s - lse


_TB = 2048


def _mlp(hp, wo, bo):
    grid = BATCH // _TB
    return pl.pallas_call(
        _mlp_body,
        grid=(grid,),
        in_specs=[
            pl.BlockSpec((NSLAB, _TB, 128), lambda i: (0, i, 0)),
            pl.BlockSpec((EMB, OUT_DIM), lambda i: (0, 0)),
            pl.BlockSpec((1, OUT_DIM), lambda i: (0, 0)),
        ],
        out_specs=pl.BlockSpec((_TB, OUT_DIM), lambda i: (i, 0)),
        out_shape=jax.ShapeDtypeStruct((BATCH, OUT_DIM), jnp.float32),
    )(hp, wo, bo)


def kernel(x, table, W_h, b_h, W_o, b_o):
    xi = x.astype(jnp.int32)
    idx0 = xi[:, 0]
    idx1 = xi[:, 1]
    tabs = _precompute(table, W_h, b_h.reshape(1, EMB))
    hp = _sc_gather(*tabs, idx0, idx1)
    hp3 = hp.reshape(NSLAB, BATCH, 128)
    return _mlp(hp3, W_o, b_o.reshape(1, OUT_DIM))
